# baseline (speedup 1.0000x reference)
import functools

import jax
import jax.numpy as jnp
from jax import lax
from jax.experimental import pallas as pl
from jax.experimental.pallas import tpu as pltpu
from jax.experimental.pallas import tpu_sc as plsc

NUM_CLASSES = 21
TOP_K = 200
CONF_THRESH = 0.01
NMS_THRESH = 0.45
NEG = -1e30

L = 16
B = 8
N = 5000
NP = 5024
NCH = NP // L
CMP = 320
KP = 256
FG = NUM_CLASSES - 1
NWORK = 32
PPW = (B * FG) // NWORK
GPB = FG // PPW
STEPS_PAD = 208
OUTW = 1040
BIGF = 1e30


def _nms_body(conf_hbm, boxes_hbm, out_hbm, s_v, x1_v, y1_v, x2_v, y2_v,
              ar_v, cm_v, cm2_v, kx1_v, ky1_v, kx2_v, ky2_v, kar_v, out_v):
    wid = lax.axis_index("c") * 16 + lax.axis_index("s")
    b = wid // GPB
    grp = wid % GPB

    lane = lax.iota(jnp.int32, L)
    lane_f = lane.astype(jnp.float32)
    negv = jnp.full((L,), NEG, jnp.float32)
    falsev = jnp.zeros((L,), jnp.bool_)
    lane_is = [lane == j for j in range(5)]
    lane0 = lane_is[0]

    pltpu.sync_copy(boxes_hbm.at[b, 0], x1_v)
    pltpu.sync_copy(boxes_hbm.at[b, 1], y1_v)
    pltpu.sync_copy(boxes_hbm.at[b, 2], x2_v)
    pltpu.sync_copy(boxes_hbm.at[b, 3], y2_v)

    @plsc.parallel_loop(0, NCH, 1, unroll=4)
    def _area_pass(i):
        sl = pl.ds(i * L, L)
        ar_v[sl] = (x2_v[sl] - x1_v[sl]) * (y2_v[sl] - y1_v[sl])

    for k in range(PPW):
        cidx = grp * PPW + k
        pltpu.sync_copy(conf_hbm.at[b, cidx], s_v)

        for i in range(KP // L):
            sl = pl.ds(i * L, L)
            kx1_v[sl] = jnp.full((L,), 2.0)
            ky1_v[sl] = jnp.full((L,), 2.0)
            kx2_v[sl] = jnp.full((L,), 2.0)
            ky2_v[sl] = jnp.full((L,), 2.0)
            kar_v[sl] = jnp.zeros((L,), jnp.float32)
        for i in range(CMP // L):
            cm_v[pl.ds(i * L, L)] = negv

        @plsc.parallel_loop(0, NCH, 1, unroll=4,
                            carry=jnp.full((L,), BIGF))
        def fi(c, fi_c):
            sl = pl.ds(c * L, L)
            v = s_v[sl]
            valid = v > CONF_THRESH
            s0 = jnp.where(valid, v, negv)
            s_v[sl] = s0
            cm = jnp.max(s0)
            plsc.store_scatter(cm_v, [jnp.full((L,), c)],
                               jnp.full((L,), cm), mask=lane0)
            g_f = (c * L).astype(jnp.float32) + lane_f
            return jnp.minimum(fi_c, jnp.where(valid, g_f, BIGF))

        for h in range(2):
            g2 = negv
            for j in range(L):
                g = h * L + j
                if g >= CMP // L:
                    break
                gmax = jnp.max(cm_v[pl.ds(g * L, L)])
                g2 = jnp.where(lane == j, gmax, g2)
            cm2_v[pl.ds(h * L, L)] = g2

        first_idx = -jnp.max(-fi)
        any_valid = first_idx < BIGF / 2
        avm = jnp.full((L,), any_valid)
        safe_fi = jnp.full(
            (L,), jnp.where(any_valid, first_idx, 0.0).astype(jnp.int32))
        pbx1 = jnp.where(avm, plsc.load_gather(x1_v, [safe_fi]), 0.0)
        pby1 = jnp.where(avm, plsc.load_gather(y1_v, [safe_fi]), 0.0)
        pbx2 = jnp.where(avm, plsc.load_gather(x2_v, [safe_fi]), 0.0)
        pby2 = jnp.where(avm, plsc.load_gather(y2_v, [safe_fi]), 0.0)

        padpats = []
        for ph in range(5):
            modv = (lane + ph * L) % 5
            pat = jnp.zeros((L,), jnp.float32)
            for coord, val in ((1, pbx1), (2, pby1), (3, pbx2), (4, pby2)):
                pat = jnp.where(modv == coord, val, pat)
            padpats.append(pat)

        def fill_pass(i, carry):
            for ph in range(5):
                out_v[pl.ds((i * 5 + ph) * L, L)] = padpats[ph]
            return carry

        lax.fori_loop(0, OUTW // (5 * L), fill_pass, 0)

        def pop_cond(carry):
            t, exhausted = carry
            return (t < TOP_K) & jnp.logical_not(exhausted)

        def pop_body(carry):
            t, _ = carry
            g2a = cm2_v[pl.ds(0, L)]
            g2b = cm2_v[pl.ds(L, L)]
            m = jnp.max(jnp.maximum(g2a, g2b))
            ok = m > NEG / 2
            f0 = plsc.all_reduce_ffs(g2a == m)[0]
            f1 = plsc.all_reduce_ffs(g2b == m)[0]
            grp = jnp.where(f0 < L, f0, f1 + L)
            gsafe = jnp.where(ok, grp, 0)
            cmv = cm_v[pl.ds(gsafe * L, L)]
            clane = plsc.all_reduce_ffs(cmv == m)[0]
            csafe = jnp.where(ok, gsafe * L + clane, 0)
            sv = s_v[pl.ds(csafe * L, L)]
            lidx = jnp.where(ok, plsc.all_reduce_ffs(sv == m)[0], 0)
            gidx = csafe * L + lidx

            sv2 = jnp.where(lane == lidx, negv, sv)
            s_v[pl.ds(csafe * L, L)] = sv2
            newmax = jnp.max(sv2)
            plsc.store_scatter(cm_v, [jnp.full((L,), csafe)],
                               jnp.full((L,), newmax), mask=lane0)
            cmv2 = jnp.where(lane == clane, newmax, cmv)
            plsc.store_scatter(cm2_v, [jnp.full((L,), gsafe)],
                               jnp.full((L,), jnp.max(cmv2)), mask=lane0)

            gv = jnp.full((L,), gidx)
            bx1 = plsc.load_gather(x1_v, [gv])
            by1 = plsc.load_gather(y1_v, [gv])
            bx2 = plsc.load_gather(x2_v, [gv])
            by2 = plsc.load_gather(y2_v, [gv])
            aC = plsc.load_gather(ar_v, [gv])

            nk = (t + (L - 1)) // L

            @plsc.parallel_loop(0, nk, 1, unroll=4, carry=falsev)
            def killv(i, acc):
                sl = pl.ds(i * L, L)
                ix1 = jnp.maximum(kx1_v[sl], bx1)
                iy1 = jnp.maximum(ky1_v[sl], by1)
                ix2 = jnp.minimum(kx2_v[sl], bx2)
                iy2 = jnp.minimum(ky2_v[sl], by2)
                inter = (jnp.maximum(ix2 - ix1, 0.0)
                         * jnp.maximum(iy2 - iy1, 0.0))
                iou = inter / ((kar_v[sl] + aC) - inter)
                return acc | (iou > NMS_THRESH)
            killed = plsc.all_reduce_population_count(killv)[0] > 0

            sel = ok & jnp.logical_not(killed)
            selv = jnp.full((L,), sel)
            tv = jnp.full((L,), t)
            selm = lane0 & selv
            plsc.store_scatter(kx1_v, [tv], bx1, mask=selm)
            plsc.store_scatter(ky1_v, [tv], by1, mask=selm)
            plsc.store_scatter(kx2_v, [tv], bx2, mask=selm)
            plsc.store_scatter(ky2_v, [tv], by2, mask=selm)
            plsc.store_scatter(kar_v, [tv], aC, mask=selm)
            row = jnp.where(lane_is[0], m,
                   jnp.where(lane_is[1], bx1,
                    jnp.where(lane_is[2], by1,
                     jnp.where(lane_is[3], bx2,
                      jnp.where(lane_is[4], by2, 0.0)))))
            plsc.store_scatter(out_v, [t * 5 + lane], row,
                               mask=selv & (lane < 5))
            return t + sel.astype(jnp.int32), jnp.logical_not(ok)

        lax.while_loop(pop_cond, pop_body, (jnp.int32(0), False))

        p = b * FG + cidx
        pltpu.sync_copy(out_v, out_hbm.at[p])


@jax.jit
def kernel(loc_data, conf_data, prior_data):
    del prior_data
    loc = loc_data.reshape(B, N, 4)
    conf = conf_data.reshape(B, N, NUM_CLASSES)
    conf_t = jnp.transpose(conf, (0, 2, 1))[:, 1:, :]
    conf_t = jnp.pad(conf_t, ((0, 0), (0, 0), (0, NP - N)))
    boxes_t = jnp.transpose(loc, (0, 2, 1))
    boxes_t = jnp.pad(boxes_t, ((0, 0), (0, 0), (0, NP - N)))

    mesh = plsc.VectorSubcoreMesh(core_axis_name="c", subcore_axis_name="s",
                                  num_cores=2, num_subcores=16)
    nms = pl.kernel(
        _nms_body,
        out_type=jax.ShapeDtypeStruct((B * FG, OUTW), jnp.float32),
        mesh=mesh,
        compiler_params=pltpu.CompilerParams(needs_layout_passes=False),
        scratch_types=[
            pltpu.VMEM((NP,), jnp.float32),
            pltpu.VMEM((NP,), jnp.float32),
            pltpu.VMEM((NP,), jnp.float32),
            pltpu.VMEM((NP,), jnp.float32),
            pltpu.VMEM((NP,), jnp.float32),
            pltpu.VMEM((NP,), jnp.float32),
            pltpu.VMEM((CMP,), jnp.float32),
            pltpu.VMEM((2 * L,), jnp.float32),
            pltpu.VMEM((KP,), jnp.float32),
            pltpu.VMEM((KP,), jnp.float32),
            pltpu.VMEM((KP,), jnp.float32),
            pltpu.VMEM((KP,), jnp.float32),
            pltpu.VMEM((KP,), jnp.float32),
            pltpu.VMEM((OUTW,), jnp.float32),
        ],
    )
    rows = nms(conf_t, boxes_t)
    rows = rows[:, :TOP_K * 5].reshape(B, FG, TOP_K, 5)
    out = jnp.concatenate(
        [jnp.zeros((B, 1, TOP_K, 5), jnp.float32), rows], axis=1)
    return out


# speedup vs baseline: 1.1170x; 1.1170x over previous
import functools

import jax
import jax.numpy as jnp
from jax import lax
from jax.experimental import pallas as pl
from jax.experimental.pallas import tpu as pltpu
from jax.experimental.pallas import tpu_sc as plsc

NUM_CLASSES = 21
TOP_K = 200
CONF_THRESH = 0.01
NMS_THRESH = 0.45
NEG = -1e30

L = 16
B = 8
N = 5000
NP = 5024
NCH = NP // L
CMP = 320
KP = 256
FG = NUM_CLASSES - 1
NWORK = 32
PPW = (B * FG) // NWORK
GPB = FG // PPW
STEPS_PAD = 208
OUTW = 1040
BIGF = 1e30


def _nms_body(conf_hbm, boxes_hbm, out_hbm, s_v, x1_v, y1_v, x2_v, y2_v,
              ar_v, cm_v, cm2_v, kx1_v, ky1_v, kx2_v, ky2_v, kar_v, out_v):
    wid = lax.axis_index("c") * 16 + lax.axis_index("s")
    b = wid // GPB
    grp = wid % GPB

    lane = lax.iota(jnp.int32, L)
    lane_f = lane.astype(jnp.float32)
    negv = jnp.full((L,), NEG, jnp.float32)
    falsev = jnp.zeros((L,), jnp.bool_)
    lane_is = [lane == j for j in range(5)]
    lane0 = lane_is[0]

    pltpu.sync_copy(boxes_hbm.at[b, 0], x1_v)
    pltpu.sync_copy(boxes_hbm.at[b, 1], y1_v)
    pltpu.sync_copy(boxes_hbm.at[b, 2], x2_v)
    pltpu.sync_copy(boxes_hbm.at[b, 3], y2_v)

    @plsc.parallel_loop(0, NCH, 1, unroll=4)
    def _area_pass(i):
        sl = pl.ds(i * L, L)
        ar_v[sl] = (x2_v[sl] - x1_v[sl]) * (y2_v[sl] - y1_v[sl])

    for k in range(PPW):
        cidx = grp * PPW + k
        pltpu.sync_copy(conf_hbm.at[b, cidx], s_v)

        for i in range(KP // L):
            sl = pl.ds(i * L, L)
            kx1_v[sl] = jnp.full((L,), 2.0)
            ky1_v[sl] = jnp.full((L,), 2.0)
            kx2_v[sl] = jnp.full((L,), 2.0)
            ky2_v[sl] = jnp.full((L,), 2.0)
            kar_v[sl] = jnp.zeros((L,), jnp.float32)
        for i in range(CMP // L):
            cm_v[pl.ds(i * L, L)] = negv

        @plsc.parallel_loop(0, NCH, 1, unroll=4,
                            carry=jnp.full((L,), BIGF))
        def fi(c, fi_c):
            sl = pl.ds(c * L, L)
            v = s_v[sl]
            valid = v > CONF_THRESH
            s0 = jnp.where(valid, v, negv)
            s_v[sl] = s0
            cm = jnp.max(s0)
            plsc.store_scatter(cm_v, [jnp.full((L,), c)],
                               jnp.full((L,), cm), mask=lane0)
            g_f = (c * L).astype(jnp.float32) + lane_f
            return jnp.minimum(fi_c, jnp.where(valid, g_f, BIGF))

        for h in range(2):
            g2 = negv
            for j in range(L):
                g = h * L + j
                if g >= CMP // L:
                    break
                gmax = jnp.max(cm_v[pl.ds(g * L, L)])
                g2 = jnp.where(lane == j, gmax, g2)
            cm2_v[pl.ds(h * L, L)] = g2

        first_idx = -jnp.max(-fi)
        any_valid = first_idx < BIGF / 2
        avm = jnp.full((L,), any_valid)
        safe_fi = jnp.full(
            (L,), jnp.where(any_valid, first_idx, 0.0).astype(jnp.int32))
        pbx1 = jnp.where(avm, plsc.load_gather(x1_v, [safe_fi]), 0.0)
        pby1 = jnp.where(avm, plsc.load_gather(y1_v, [safe_fi]), 0.0)
        pbx2 = jnp.where(avm, plsc.load_gather(x2_v, [safe_fi]), 0.0)
        pby2 = jnp.where(avm, plsc.load_gather(y2_v, [safe_fi]), 0.0)

        padpats = []
        for ph in range(5):
            modv = (lane + ph * L) % 5
            pat = jnp.zeros((L,), jnp.float32)
            for coord, val in ((1, pbx1), (2, pby1), (3, pbx2), (4, pby2)):
                pat = jnp.where(modv == coord, val, pat)
            padpats.append(pat)

        def fill_pass(i, carry):
            for ph in range(5):
                out_v[pl.ds((i * 5 + ph) * L, L)] = padpats[ph]
            return carry

        lax.fori_loop(0, OUTW // (5 * L), fill_pass, 0)

        def pop_cond(carry):
            t, exhausted = carry
            return (t < TOP_K) & jnp.logical_not(exhausted)

        def pop_body(carry):
            t, _ = carry
            g2a = cm2_v[pl.ds(0, L)]
            g2b = cm2_v[pl.ds(L, L)]
            m = jnp.max(jnp.maximum(g2a, g2b))
            ok = m > NEG / 2
            f0 = plsc.all_reduce_ffs(g2a == m)[0]
            f1 = plsc.all_reduce_ffs(g2b == m)[0]
            grp = jnp.where(f0 < L, f0, f1 + L)
            gsafe = jnp.where(ok, grp, 0)
            cmv = cm_v[pl.ds(gsafe * L, L)]
            clane = plsc.all_reduce_ffs(cmv == m)[0]
            csafe = jnp.where(ok, gsafe * L + clane, 0)
            sv = s_v[pl.ds(csafe * L, L)]
            lidx = jnp.where(ok, plsc.all_reduce_ffs(sv == m)[0], 0)
            gidx = csafe * L + lidx

            sv2 = jnp.where(lane == lidx, negv, sv)
            s_v[pl.ds(csafe * L, L)] = sv2
            newmax = jnp.max(sv2)
            plsc.store_scatter(cm_v, [jnp.full((L,), csafe)],
                               jnp.full((L,), newmax), mask=lane0)
            cmv2 = jnp.where(lane == clane, newmax, cmv)
            plsc.store_scatter(cm2_v, [jnp.full((L,), gsafe)],
                               jnp.full((L,), jnp.max(cmv2)), mask=lane0)

            gv = jnp.full((L,), gidx)
            bx1 = plsc.load_gather(x1_v, [gv])
            by1 = plsc.load_gather(y1_v, [gv])
            bx2 = plsc.load_gather(x2_v, [gv])
            by2 = plsc.load_gather(y2_v, [gv])
            aC = plsc.load_gather(ar_v, [gv])

            def kept_chunk(i, acc):
                for u in range(4):
                    sl = pl.ds((i * 4 + u) * L, L)
                    ix1 = jnp.maximum(kx1_v[sl], bx1)
                    iy1 = jnp.maximum(ky1_v[sl], by1)
                    ix2 = jnp.minimum(kx2_v[sl], bx2)
                    iy2 = jnp.minimum(ky2_v[sl], by2)
                    inter = (jnp.maximum(ix2 - ix1, 0.0)
                             * jnp.maximum(iy2 - iy1, 0.0))
                    iou = inter / ((kar_v[sl] + aC) - inter)
                    acc = acc | (iou > NMS_THRESH)
                return acc

            nk = (t + (4 * L - 1)) // (4 * L)
            killv = lax.fori_loop(0, nk, kept_chunk, falsev)
            killed = plsc.all_reduce_population_count(killv)[0] > 0

            sel = ok & jnp.logical_not(killed)
            selv = jnp.full((L,), sel)
            tv = jnp.full((L,), t)
            selm = lane0 & selv
            plsc.store_scatter(kx1_v, [tv], bx1, mask=selm)
            plsc.store_scatter(ky1_v, [tv], by1, mask=selm)
            plsc.store_scatter(kx2_v, [tv], bx2, mask=selm)
            plsc.store_scatter(ky2_v, [tv], by2, mask=selm)
            plsc.store_scatter(kar_v, [tv], aC, mask=selm)
            row = jnp.where(lane_is[0], m,
                   jnp.where(lane_is[1], bx1,
                    jnp.where(lane_is[2], by1,
                     jnp.where(lane_is[3], bx2,
                      jnp.where(lane_is[4], by2, 0.0)))))
            plsc.store_scatter(out_v, [t * 5 + lane], row,
                               mask=selv & (lane < 5))
            return t + sel.astype(jnp.int32), jnp.logical_not(ok)

        lax.while_loop(pop_cond, pop_body, (jnp.int32(0), False))

        p = b * FG + cidx
        pltpu.sync_copy(out_v, out_hbm.at[p])


@jax.jit
def kernel(loc_data, conf_data, prior_data):
    del prior_data
    loc = loc_data.reshape(B, N, 4)
    conf = conf_data.reshape(B, N, NUM_CLASSES)
    conf_t = jnp.transpose(conf, (0, 2, 1))[:, 1:, :]
    conf_t = jnp.pad(conf_t, ((0, 0), (0, 0), (0, NP - N)))
    boxes_t = jnp.transpose(loc, (0, 2, 1))
    boxes_t = jnp.pad(boxes_t, ((0, 0), (0, 0), (0, NP - N)))

    mesh = plsc.VectorSubcoreMesh(core_axis_name="c", subcore_axis_name="s",
                                  num_cores=2, num_subcores=16)
    nms = pl.kernel(
        _nms_body,
        out_type=jax.ShapeDtypeStruct((B * FG, OUTW), jnp.float32),
        mesh=mesh,
        compiler_params=pltpu.CompilerParams(needs_layout_passes=False),
        scratch_types=[
            pltpu.VMEM((NP,), jnp.float32),
            pltpu.VMEM((NP,), jnp.float32),
            pltpu.VMEM((NP,), jnp.float32),
            pltpu.VMEM((NP,), jnp.float32),
            pltpu.VMEM((NP,), jnp.float32),
            pltpu.VMEM((NP,), jnp.float32),
            pltpu.VMEM((CMP,), jnp.float32),
            pltpu.VMEM((2 * L,), jnp.float32),
            pltpu.VMEM((KP,), jnp.float32),
            pltpu.VMEM((KP,), jnp.float32),
            pltpu.VMEM((KP,), jnp.float32),
            pltpu.VMEM((KP,), jnp.float32),
            pltpu.VMEM((KP,), jnp.float32),
            pltpu.VMEM((OUTW,), jnp.float32),
        ],
    )
    rows = nms(conf_t, boxes_t)
    rows = rows[:, :TOP_K * 5].reshape(B, FG, TOP_K, 5)
    out = jnp.concatenate(
        [jnp.zeros((B, 1, TOP_K, 5), jnp.float32), rows], axis=1)
    return out
